# baseline (speedup 1.0000x reference)
import functools

import jax
import jax.numpy as jnp
import numpy as np
from jax.experimental import pallas as pl

_C = 768
_C2 = _C // 2
_C4 = _C // 4
_BN = 256
_EPS = 1e-5


def _erfc_xla(x):
    ax = jnp.abs(x)
    x2 = x * x
    p = x2 * np.float32(7.85386146e-05) + np.float32(-0.000801019371)
    p = p * x2 + np.float32(0.00518832775)
    p = p * x2 + np.float32(-0.0268538129)
    p = p * x2 + np.float32(0.112835854)
    p = p * x2 + np.float32(-0.37612626)
    p = p * x2 + np.float32(1.12837911)
    b1 = np.float32(1.0) - x * p
    q = np.float32(1.0) / x2
    zr = jnp.exp(-x2) * (np.float32(1.0) / ax)
    p1 = q * np.float32(0.0232682) + np.float32(-0.138703942)
    p1 = p1 * q + np.float32(0.368742466)
    p1 = p1 * q + np.float32(-0.582473278)
    p1 = p1 * q + np.float32(0.621000469)
    p1 = p1 * q + np.float32(-0.494451523)
    p1 = p1 * q + np.float32(0.340488)
    p1 = p1 * q + np.float32(-0.274112701)
    p1 = p1 * q + np.float32(0.563825965)
    p2 = q * np.float32(-10.477664) + np.float32(12.9772)
    p2 = p2 * q + np.float32(-7.49551868)
    p2 = p2 * q + np.float32(2.92101908)
    p2 = p2 * q + np.float32(-1.01526523)
    p2 = p2 * q + np.float32(0.42184633)
    p2 = p2 * q + np.float32(-0.282076746)
    p2 = p2 * q + np.float32(0.564189494)
    y = zr * jnp.where(ax < np.float32(2.0), p1, p2)
    y = jnp.where(-x2 < np.float32(-88.7228394), np.float32(0.0), y)
    y = jnp.where(x < np.float32(0.0), np.float32(2.0) - y, y)
    return jnp.where(ax < np.float32(1.0), b1, y)


_SQRT_HALF = np.float32(np.sqrt(0.5))


def _gelu(x):
    return np.float32(0.5) * x * _erfc_xla(-x * _SQRT_HALF)


def _bdot(a, b):
    return jnp.dot(a.astype(jnp.bfloat16), b.astype(jnp.bfloat16),
                   preferred_element_type=jnp.float32)


def _pass1_body(x_ref, m_ref, v_ref, w_ref, gam_ref, bet_ref, W1_ref, b1_ref,
                xloc_ref, gsum_ref, wsum_ref):
    j = pl.program_id(1)
    x = x_ref[0]
    xn = (x - m_ref[0]) / jnp.sqrt(v_ref[0] + _EPS) * gam_ref[0] + bet_ref[0]
    y = _gelu(_bdot(xn, W1_ref[...]) + b1_ref[0])
    xloc_ref[0] = y[:, :_C2]
    w = w_ref[0]
    part = jnp.sum(y[:, _C2:] * w, axis=0).reshape(1, 1, _C2)
    pwv = jnp.full((1, 1, 128), jnp.sum(w), jnp.float32)

    @pl.when(j == 0)
    def _init():
        gsum_ref[...] = part
        wsum_ref[...] = pwv

    @pl.when(j != 0)
    def _acc():
        gsum_ref[...] += part
        wsum_ref[...] += pwv


def _pass2_body(xloc_ref, gsum_ref, wsum_ref, noise_ref, g_ref, w_ref,
                W2a_ref, W2b_ref, W2c_ref, b2_ref, W3_ref, b3_ref,
                W4_ref, b4_ref, out_ref):
    gx = gsum_ref[0] / wsum_ref[0][:, :1]
    cbias = (_bdot(gx, W2b_ref[...]) + _bdot(noise_ref[0], W2c_ref[...])
             + b2_ref[0])
    h = _gelu(_bdot(xloc_ref[0], W2a_ref[...]) + cbias)
    h = _gelu(_bdot(h, W3_ref[...]) + b3_ref[0])
    logits = _bdot(h, W4_ref[...]) + b4_ref[0]
    l0 = logits[:, 0:1]
    l1 = logits[:, 1:2]
    mx = jnp.maximum(l0, l1)
    s0 = l0 - mx
    s1 = l1 - mx
    lse = jnp.log(jnp.exp(s0) + jnp.exp(s1))
    g = g_ref[0]
    a0 = (s0 - lse) + g[:, 0:1]
    a1 = (s1 - lse) + g[:, 1:2]
    amx = jnp.maximum(a0, a1)
    e0 = jnp.exp(a0 - amx)
    e1 = jnp.exp(a1 - amx)
    s = e0 + e1
    out_ref[0] = jnp.where(e0 / s >= e1 / s, w_ref[0], 0.0)


@jax.jit
def kernel(input_feature, noise_feature, prev_m, ln_gamma, ln_beta,
           W1, b1, W2, b2, W3, b3, W4, b4):
    B, N, C = input_feature.shape
    nskip = N - prev_m.shape[1]
    NB = N // _BN

    xs = input_feature[:, nskip:, :]
    m = jnp.mean(xs, axis=-1, keepdims=True)
    v = jnp.mean((xs - m) ** 2, axis=-1, keepdims=True)
    m = jnp.pad(m, ((0, 0), (nskip, 0), (0, 0)))
    v = jnp.pad(v, ((0, 0), (nskip, 0), (0, 0)), constant_values=1.0)

    g = jax.random.gumbel(jax.random.key(42), (B, N - nskip, 2), jnp.float32)
    g = jnp.pad(g, ((0, 0), (nskip, 0), (0, 0)))
    w = jnp.pad(prev_m, ((0, 0), (nskip, 0), (0, 0)))

    xloc, gsum, wsum = pl.pallas_call(
        _pass1_body,
        grid=(B, NB),
        in_specs=[
            pl.BlockSpec((1, _BN, C), lambda i, j: (i, j, 0)),
            pl.BlockSpec((1, _BN, 1), lambda i, j: (i, j, 0)),
            pl.BlockSpec((1, _BN, 1), lambda i, j: (i, j, 0)),
            pl.BlockSpec((1, _BN, 1), lambda i, j: (i, j, 0)),
            pl.BlockSpec((1, C), lambda i, j: (0, 0)),
            pl.BlockSpec((1, C), lambda i, j: (0, 0)),
            pl.BlockSpec((C, C), lambda i, j: (0, 0)),
            pl.BlockSpec((1, C), lambda i, j: (0, 0)),
        ],
        out_specs=[
            pl.BlockSpec((1, _BN, _C2), lambda i, j: (i, j, 0)),
            pl.BlockSpec((1, 1, _C2), lambda i, j: (i, 0, 0)),
            pl.BlockSpec((1, 1, 128), lambda i, j: (i, 0, 0)),
        ],
        out_shape=[
            jax.ShapeDtypeStruct((B, N, _C2), jnp.float32),
            jax.ShapeDtypeStruct((B, 1, _C2), jnp.float32),
            jax.ShapeDtypeStruct((B, 1, 128), jnp.float32),
        ],
    )(input_feature, m, v, w, ln_gamma.reshape(1, C), ln_beta.reshape(1, C),
      W1, b1.reshape(1, C))

    m_full = pl.pallas_call(
        _pass2_body,
        grid=(B, NB),
        in_specs=[
            pl.BlockSpec((1, _BN, _C2), lambda i, j: (i, j, 0)),
            pl.BlockSpec((1, 1, _C2), lambda i, j: (i, 0, 0)),
            pl.BlockSpec((1, 1, 128), lambda i, j: (i, 0, 0)),
            pl.BlockSpec((1, 1, _C2), lambda i, j: (i, 0, 0)),
            pl.BlockSpec((1, _BN, 2), lambda i, j: (i, j, 0)),
            pl.BlockSpec((1, _BN, 1), lambda i, j: (i, j, 0)),
            pl.BlockSpec((_C2, _C2), lambda i, j: (0, 0)),
            pl.BlockSpec((_C2, _C2), lambda i, j: (0, 0)),
            pl.BlockSpec((_C2, _C2), lambda i, j: (0, 0)),
            pl.BlockSpec((1, _C2), lambda i, j: (0, 0)),
            pl.BlockSpec((_C2, _C4), lambda i, j: (0, 0)),
            pl.BlockSpec((1, _C4), lambda i, j: (0, 0)),
            pl.BlockSpec((_C4, 2), lambda i, j: (0, 0)),
            pl.BlockSpec((1, 2), lambda i, j: (0, 0)),
        ],
        out_specs=pl.BlockSpec((1, _BN, 1), lambda i, j: (i, j, 0)),
        out_shape=jax.ShapeDtypeStruct((B, N, 1), jnp.float32),
    )(xloc, gsum, wsum, noise_feature, g, w,
      W2[:_C2], W2[_C2:2 * _C2], W2[2 * _C2:], b2.reshape(1, _C2),
      W3, b3.reshape(1, _C4), W4, b4.reshape(1, 2))

    curr_m = m_full[:, nskip:, :]
    return (input_feature, curr_m)


# speedup vs baseline: 1.1210x; 1.1210x over previous
import functools

import jax
import jax.numpy as jnp
import numpy as np
from jax.experimental import pallas as pl

_C = 768
_C2 = _C // 2
_C4 = _C // 4
_BN = 512
_EPS = 1e-5


def _erfc_xla(x):
    ax = jnp.abs(x)
    x2 = x * x
    p = x2 * np.float32(7.85386146e-05) + np.float32(-0.000801019371)
    p = p * x2 + np.float32(0.00518832775)
    p = p * x2 + np.float32(-0.0268538129)
    p = p * x2 + np.float32(0.112835854)
    p = p * x2 + np.float32(-0.37612626)
    p = p * x2 + np.float32(1.12837911)
    b1 = np.float32(1.0) - x * p
    q = np.float32(1.0) / x2
    zr = jnp.exp(-x2) * (np.float32(1.0) / ax)
    p1 = q * np.float32(0.0232682) + np.float32(-0.138703942)
    p1 = p1 * q + np.float32(0.368742466)
    p1 = p1 * q + np.float32(-0.582473278)
    p1 = p1 * q + np.float32(0.621000469)
    p1 = p1 * q + np.float32(-0.494451523)
    p1 = p1 * q + np.float32(0.340488)
    p1 = p1 * q + np.float32(-0.274112701)
    p1 = p1 * q + np.float32(0.563825965)
    p2 = q * np.float32(-10.477664) + np.float32(12.9772)
    p2 = p2 * q + np.float32(-7.49551868)
    p2 = p2 * q + np.float32(2.92101908)
    p2 = p2 * q + np.float32(-1.01526523)
    p2 = p2 * q + np.float32(0.42184633)
    p2 = p2 * q + np.float32(-0.282076746)
    p2 = p2 * q + np.float32(0.564189494)
    y = zr * jnp.where(ax < np.float32(2.0), p1, p2)
    y = jnp.where(-x2 < np.float32(-88.7228394), np.float32(0.0), y)
    y = jnp.where(x < np.float32(0.0), np.float32(2.0) - y, y)
    return jnp.where(ax < np.float32(1.0), b1, y)


_SQRT_HALF = np.float32(np.sqrt(0.5))


def _gelu(x):
    return np.float32(0.5) * x * _erfc_xla(-x * _SQRT_HALF)


def _bdot(a, b):
    return jnp.dot(a.astype(jnp.bfloat16), b.astype(jnp.bfloat16),
                   preferred_element_type=jnp.float32)


def _pass1_body(x_ref, m_ref, v_ref, w_ref, gam_ref, bet_ref, W1_ref, b1_ref,
                xloc_ref, gsum_ref, wsum_ref):
    j = pl.program_id(1)
    x = x_ref[0]
    xn = (x - m_ref[0]) / jnp.sqrt(v_ref[0] + _EPS) * gam_ref[0] + bet_ref[0]
    y = _gelu(_bdot(xn, W1_ref[...]) + b1_ref[0])
    xloc_ref[0] = y[:, :_C2].astype(jnp.bfloat16)
    w = w_ref[0]
    part = jnp.sum(y[:, _C2:] * w, axis=0).reshape(1, 1, _C2)
    pwv = jnp.full((1, 1, 128), jnp.sum(w), jnp.float32)

    @pl.when(j == 0)
    def _init():
        gsum_ref[...] = part
        wsum_ref[...] = pwv

    @pl.when(j != 0)
    def _acc():
        gsum_ref[...] += part
        wsum_ref[...] += pwv


def _cbias_body(gsum_ref, wsum_ref, noise_ref, W2b_ref, W2c_ref, b2_ref,
                cb_ref):
    gx = gsum_ref[0] / wsum_ref[0][:, :1]
    cb_ref[0] = (_bdot(gx, W2b_ref[...]) + _bdot(noise_ref[0], W2c_ref[...])
                 + b2_ref[0])


def _pass2_body(xloc_ref, cb_ref, g_ref, w_ref,
                W2a_ref, W3_ref, b3_ref, W4_ref, b4_ref, out_ref):
    h = _gelu(_bdot(xloc_ref[0], W2a_ref[...]) + cb_ref[0])
    h = _gelu(_bdot(h, W3_ref[...]) + b3_ref[0])
    logits = _bdot(h, W4_ref[...]) + b4_ref[0]
    l0 = logits[:, 0:1]
    l1 = logits[:, 1:2]
    mx = jnp.maximum(l0, l1)
    s0 = l0 - mx
    s1 = l1 - mx
    lse = jnp.log(jnp.exp(s0) + jnp.exp(s1))
    g = g_ref[0]
    a0 = (s0 - lse) + g[:, 0:1]
    a1 = (s1 - lse) + g[:, 1:2]
    amx = jnp.maximum(a0, a1)
    e0 = jnp.exp(a0 - amx)
    e1 = jnp.exp(a1 - amx)
    s = e0 + e1
    out_ref[0] = jnp.where(e0 / s >= e1 / s, w_ref[0], 0.0)


@jax.jit
def kernel(input_feature, noise_feature, prev_m, ln_gamma, ln_beta,
           W1, b1, W2, b2, W3, b3, W4, b4):
    B, N, C = input_feature.shape
    nskip = N - prev_m.shape[1]
    NB = N // _BN

    xs = input_feature[:, nskip:, :]
    m = jnp.mean(xs, axis=-1, keepdims=True)
    v = jnp.mean((xs - m) ** 2, axis=-1, keepdims=True)
    m = jnp.pad(m, ((0, 0), (nskip, 0), (0, 0)))
    v = jnp.pad(v, ((0, 0), (nskip, 0), (0, 0)), constant_values=1.0)

    g = jax.random.gumbel(jax.random.key(42), (B, N - nskip, 2), jnp.float32)
    g = jnp.pad(g, ((0, 0), (nskip, 0), (0, 0)))
    w = jnp.pad(prev_m, ((0, 0), (nskip, 0), (0, 0)))

    xloc, gsum, wsum = pl.pallas_call(
        _pass1_body,
        grid=(B, NB),
        in_specs=[
            pl.BlockSpec((1, _BN, C), lambda i, j: (i, j, 0)),
            pl.BlockSpec((1, _BN, 1), lambda i, j: (i, j, 0)),
            pl.BlockSpec((1, _BN, 1), lambda i, j: (i, j, 0)),
            pl.BlockSpec((1, _BN, 1), lambda i, j: (i, j, 0)),
            pl.BlockSpec((1, C), lambda i, j: (0, 0)),
            pl.BlockSpec((1, C), lambda i, j: (0, 0)),
            pl.BlockSpec((C, C), lambda i, j: (0, 0)),
            pl.BlockSpec((1, C), lambda i, j: (0, 0)),
        ],
        out_specs=[
            pl.BlockSpec((1, _BN, _C2), lambda i, j: (i, j, 0)),
            pl.BlockSpec((1, 1, _C2), lambda i, j: (i, 0, 0)),
            pl.BlockSpec((1, 1, 128), lambda i, j: (i, 0, 0)),
        ],
        out_shape=[
            jax.ShapeDtypeStruct((B, N, _C2), jnp.bfloat16),
            jax.ShapeDtypeStruct((B, 1, _C2), jnp.float32),
            jax.ShapeDtypeStruct((B, 1, 128), jnp.float32),
        ],
    )(input_feature, m, v, w, ln_gamma.reshape(1, C), ln_beta.reshape(1, C),
      W1, b1.reshape(1, C))

    cbias = pl.pallas_call(
        _cbias_body,
        grid=(B,),
        in_specs=[
            pl.BlockSpec((1, 1, _C2), lambda i: (i, 0, 0)),
            pl.BlockSpec((1, 1, 128), lambda i: (i, 0, 0)),
            pl.BlockSpec((1, 1, _C2), lambda i: (i, 0, 0)),
            pl.BlockSpec((_C2, _C2), lambda i: (0, 0)),
            pl.BlockSpec((_C2, _C2), lambda i: (0, 0)),
            pl.BlockSpec((1, _C2), lambda i: (0, 0)),
        ],
        out_specs=pl.BlockSpec((1, 1, _C2), lambda i: (i, 0, 0)),
        out_shape=jax.ShapeDtypeStruct((B, 1, _C2), jnp.float32),
    )(gsum, wsum, noise_feature, W2[_C2:2 * _C2], W2[2 * _C2:],
      b2.reshape(1, _C2))

    m_full = pl.pallas_call(
        _pass2_body,
        grid=(B, NB),
        in_specs=[
            pl.BlockSpec((1, _BN, _C2), lambda i, j: (i, j, 0)),
            pl.BlockSpec((1, 1, _C2), lambda i, j: (i, 0, 0)),
            pl.BlockSpec((1, _BN, 2), lambda i, j: (i, j, 0)),
            pl.BlockSpec((1, _BN, 1), lambda i, j: (i, j, 0)),
            pl.BlockSpec((_C2, _C2), lambda i, j: (0, 0)),
            pl.BlockSpec((_C2, _C4), lambda i, j: (0, 0)),
            pl.BlockSpec((1, _C4), lambda i, j: (0, 0)),
            pl.BlockSpec((_C4, 2), lambda i, j: (0, 0)),
            pl.BlockSpec((1, 2), lambda i, j: (0, 0)),
        ],
        out_specs=pl.BlockSpec((1, _BN, 1), lambda i, j: (i, j, 0)),
        out_shape=jax.ShapeDtypeStruct((B, N, 1), jnp.float32),
    )(xloc, cbias, g, w, W2[:_C2], W3, b3.reshape(1, _C4), W4,
      b4.reshape(1, 2))

    curr_m = m_full[:, nskip:, :]
    return (input_feature, curr_m)
